# baseline (speedup 1.0000x reference)
import functools

import numpy as np
import jax
import jax.numpy as jnp
from jax import lax
from jax.experimental import pallas as pl
from jax.experimental.pallas import tpu as pltpu
from jax.experimental.pallas import tpu_sc as plsc

_FFT_TOPK = 8
_EPS = 1e-06
_F = 64
_BB = 8
_BD = 16

_NC = 2
_NS = 16
_L = 16


def _dft_consts():
    n = np.arange(_F)
    ang = 2.0 * np.pi * np.outer(n, n) / _F
    c64 = np.cos(ang)
    s64 = np.sin(ang)
    angt = 2.0 * np.pi * np.outer(n, n) / (_F * _F)
    twc = np.cos(angt)
    tws = np.sin(angt)
    return (c64.astype(np.float32), s64.astype(np.float32),
            twc.astype(np.float32), tws.astype(np.float32))


_C64, _S64, _TWC, _TWS = _dft_consts()


def _dft_body(xr0_ref, c_ref, s_ref, twc_ref, tws_ref, mag_ref):
    T = _F * _F
    f32 = jnp.float32
    bf16 = jnp.bfloat16
    cm = c_ref[...].astype(bf16)
    sm = s_ref[...].astype(bf16)
    twc = twc_ref[...]
    tws = tws_ref[...]
    k1 = lax.broadcasted_iota(jnp.int32, (_F, _F), 0)
    k2 = lax.broadcasted_iota(jnp.int32, (_F, _F), 1)
    kmask = (k1 + _F * k2 >= 1) & (k1 + _F * k2 <= T // 2)
    ars, ais = [], []
    for i in range(_BD):
        xr = xr0_ref[i].astype(bf16)
        ars.append(jnp.dot(cm, xr, preferred_element_type=f32))
        ais.append(-jnp.dot(sm, xr, preferred_element_type=f32))
    for i in range(_BD):
        br = (ars[i] * twc + ais[i] * tws).astype(bf16)
        bi = (ais[i] * twc - ars[i] * tws).astype(bf16)
        xre = jnp.dot(br, cm, preferred_element_type=f32) + jnp.dot(bi, sm, preferred_element_type=f32)
        xim = jnp.dot(bi, cm, preferred_element_type=f32) - jnp.dot(br, sm, preferred_element_type=f32)
        mag2 = xre * xre + xim * xim
        mag_ref[i] = jnp.where(kmask, mag2, -1.0)


def _dft_mag2(xr0):
    B = xr0.shape[0]
    return pl.pallas_call(
        _dft_body,
        grid=(B // _BD,),
        in_specs=[
            pl.BlockSpec((_BD, _F, _F), lambda b: (b, 0, 0)),
            pl.BlockSpec((_F, _F), lambda b: (0, 0)),
            pl.BlockSpec((_F, _F), lambda b: (0, 0)),
            pl.BlockSpec((_F, _F), lambda b: (0, 0)),
            pl.BlockSpec((_F, _F), lambda b: (0, 0)),
        ],
        out_specs=pl.BlockSpec((_BD, _F, _F), lambda b: (b, 0, 0)),
        out_shape=jax.ShapeDtypeStruct((B, _F, _F), jnp.float32),
    )(xr0, _C64, _S64, _TWC, _TWS)


def _sc_topk(magm):
    B, T = magm.shape
    nw = _NC * _NS
    rows_per_w = B // nw
    nchunks = T // _L
    mesh = plsc.VectorSubcoreMesh(core_axis_name="c", subcore_axis_name="s")

    @functools.partial(
        pl.kernel,
        mesh=mesh,
        out_type=jax.ShapeDtypeStruct((B, _L), jnp.float32),
        scratch_types=[
            pltpu.VMEM((T,), jnp.float32),
            pltpu.VMEM((_L,), jnp.float32),
        ],
    )
    def topk_kernel(mag_hbm, out_hbm, row_v, val_v):
        wid = lax.axis_index("s") * _NC + lax.axis_index("c")
        lane = lax.iota(jnp.int32, _L)

        def shuffle(v, idx):
            return lax.gather(
                v, idx[:, None],
                lax.GatherDimensionNumbers(
                    offset_dims=(), collapsed_slice_dims=(0,),
                    start_index_map=(0,)),
                (1,), mode=lax.GatherScatterMode.PROMISE_IN_BOUNDS)

        def splat_max(v):
            for d in (1, 2, 4, 8):
                v = jnp.maximum(v, shuffle(v, lane ^ d))
            return v
        for r in range(rows_per_w):
            row = wid * rows_per_w + r
            pltpu.sync_copy(mag_hbm.at[row], row_v)
            vals = jnp.full((_L,), -1.0, jnp.float32)
            s_prev = jnp.full((_L,), -2.0, jnp.float32)
            for j in range(_FFT_TOPK):
                def scan(c, m):
                    v = row_v[pl.ds(c * _L, _L)]
                    v = jnp.where(v == s_prev, -1.0, v)
                    row_v[pl.ds(c * _L, _L)] = v
                    return jnp.maximum(m, v)
                m = lax.fori_loop(0, nchunks, scan,
                                  jnp.full((_L,), -2.0, jnp.float32))
                smax = splat_max(m)
                vals = jnp.where(lane == j, smax, vals)
                s_prev = smax
            val_v[...] = vals
            pltpu.sync_copy(val_v, out_hbm.at[row])

    return topk_kernel(magm)


def _stream_body(x_ref, v_ref, out_ref):
    _, C, T = x_ref.shape
    f32 = jnp.float32
    for i in range(_BB):
        xb = x_ref[i]
        s1 = jnp.sum(xb, axis=1, keepdims=True)
        s2 = jnp.sum(xb * xb, axis=1, keepdims=True)
        mean = s1 / T
        var = s2 / T - mean * mean
        std = jnp.sqrt(var)
        a = 1.0 / (std + _EPS)
        bb = -mean * a
        out_ref[i, :C, :] = xb * a + bb
        scale = 1.0 / ((std[0, 0] + _EPS) * (T + 1e-09))
        for j in range(_FFT_TOPK):
            out_ref[i, C + j, :] = jnp.full(
                (T,), jnp.sqrt(v_ref[i, j]) * scale, f32)


def kernel(x):
    B, C, T = x.shape
    xr0 = x[:, 0, :].reshape(B, _F, _F)
    magm = _dft_mag2(xr0).reshape(B, T)
    vals2 = _sc_topk(magm)
    out = pl.pallas_call(
        _stream_body,
        grid=(B // _BB,),
        in_specs=[
            pl.BlockSpec((_BB, C, T), lambda b: (b, 0, 0)),
            pl.BlockSpec((_BB, _L), lambda b: (b, 0)),
        ],
        out_specs=pl.BlockSpec((_BB, C + _FFT_TOPK, T), lambda b: (b, 0, 0)),
        out_shape=jax.ShapeDtypeStruct((B, C + _FFT_TOPK, T), jnp.float32),
    )(x, vals2)
    return out


# speedup vs baseline: 1.5766x; 1.5766x over previous
import numpy as np
import jax
import jax.numpy as jnp
from jax.experimental import pallas as pl

_FFT_TOPK = 8
_EPS = 1e-06
_F = 64
_BB = 8


def _dft_consts():
    n = np.arange(_F)
    ang = 2.0 * np.pi * np.outer(n, n) / _F
    c64 = np.cos(ang)
    s64 = np.sin(ang)
    angt = 2.0 * np.pi * np.outer(n, n) / (_F * _F)
    twc = np.cos(angt)
    tws = np.sin(angt)
    return (c64.astype(np.float32), s64.astype(np.float32),
            twc.astype(np.float32), tws.astype(np.float32))


_C64, _S64, _TWC, _TWS = _dft_consts()


def _body(x_ref, xr0_ref, ones_ref, c_ref, s_ref, twc_ref, tws_ref, out_ref):
    _, C, T = x_ref.shape
    f32 = jnp.float32
    bf16 = jnp.bfloat16
    cm = c_ref[...].astype(bf16)
    sm = s_ref[...].astype(bf16)
    twc = twc_ref[...]
    tws = tws_ref[...]
    k1 = jax.lax.broadcasted_iota(jnp.int32, (_F, _F), 0)
    k2 = jax.lax.broadcasted_iota(jnp.int32, (_F, _F), 1)
    kmask = (k1 + _F * k2 >= 1) & (k1 + _F * k2 <= T // 2)
    ars, ais = [], []
    for i in range(_BB):
        xr = xr0_ref[i].astype(bf16)
        ars.append(jnp.dot(cm, xr, preferred_element_type=f32))
        ais.append(-jnp.dot(sm, xr, preferred_element_type=f32))

    ii = jax.lax.broadcasted_iota(jnp.int32, (64, 64), 0)
    jj = jax.lax.broadcasted_iota(jnp.int32, (64, 64), 1)
    eye = ii == jj
    stats = []
    for i in range(_BB):
        xbh = x_ref[i].astype(bf16)
        sums = jax.lax.dot_general(xbh, ones_ref[...],
                                   (((1,), (0,)), ((), ())),
                                   preferred_element_type=f32)[:, :1]
        gram = jax.lax.dot_general(xbh, xbh, (((1,), (1,)), ((), ())),
                                   preferred_element_type=f32)
        stats.append((sums, gram))

    stds = []
    for i in range(_BB):
        sums, gram = stats[i]
        sumsq = jnp.sum(jnp.where(eye, gram, 0.0), axis=1, keepdims=True)
        mean = sums / T
        var = sumsq / T - mean * mean
        std = jnp.sqrt(var)
        stds.append(std)
        a = 1.0 / (std + _EPS)
        bb = -mean * a
        out_ref[i, :C, :] = x_ref[i] * a + bb

    maskeds = []
    for i in range(_BB):
        br = (ars[i] * twc + ais[i] * tws).astype(bf16)
        bi = (ais[i] * twc - ars[i] * tws).astype(bf16)
        xre = jnp.dot(br, cm, preferred_element_type=f32) + jnp.dot(bi, sm, preferred_element_type=f32)
        xim = jnp.dot(bi, cm, preferred_element_type=f32) - jnp.dot(br, sm, preferred_element_type=f32)
        mag2 = xre * xre + xim * xim
        maskeds.append(jnp.where(kmask, mag2, -1.0))

    vals = [[] for _ in range(_BB)]
    for _ in range(_FFT_TOPK):
        for i in range(_BB):
            m = jnp.max(maskeds[i])
            vals[i].append(m)
            maskeds[i] = jnp.where(maskeds[i] == m, -1.0, maskeds[i])

    for i in range(_BB):
        scale = 1.0 / ((stds[i][0, 0] + _EPS) * (T + 1e-09))
        for j in range(_FFT_TOPK):
            out_ref[i, C + j, :] = jnp.full((T,), jnp.sqrt(vals[i][j]) * scale, f32)


def kernel(x):
    B, C, T = x.shape
    xr0 = x[:, 0, :].reshape(B, _F, _F)
    out = pl.pallas_call(
        _body,
        grid=(B // _BB,),
        in_specs=[
            pl.BlockSpec((_BB, C, T), lambda b: (b, 0, 0)),
            pl.BlockSpec((_BB, _F, _F), lambda b: (b, 0, 0)),
            pl.BlockSpec((T, 128), lambda b: (0, 0)),
            pl.BlockSpec((_F, _F), lambda b: (0, 0)),
            pl.BlockSpec((_F, _F), lambda b: (0, 0)),
            pl.BlockSpec((_F, _F), lambda b: (0, 0)),
            pl.BlockSpec((_F, _F), lambda b: (0, 0)),
        ],
        out_specs=pl.BlockSpec((_BB, C + _FFT_TOPK, T), lambda b: (b, 0, 0)),
        out_shape=jax.ShapeDtypeStruct((B, C + _FFT_TOPK, T), jnp.float32),
    )(x, xr0, jnp.ones((T, 128), jnp.bfloat16), _C64, _S64, _TWC, _TWS)
    return out


# speedup vs baseline: 1.6908x; 1.0724x over previous
import numpy as np
import jax
import jax.numpy as jnp
from jax.experimental import pallas as pl
from jax.experimental.pallas import tpu as pltpu

_FFT_TOPK = 8
_EPS = 1e-06
_F = 64
_BB = 8


def _dft_consts():
    n = np.arange(_F)
    ang = 2.0 * np.pi * np.outer(n, n) / _F
    c64 = np.cos(ang)
    s64 = np.sin(ang)
    angt = 2.0 * np.pi * np.outer(n, n) / (_F * _F)
    twc = np.cos(angt)
    tws = np.sin(angt)
    return (c64.astype(np.float32), s64.astype(np.float32),
            twc.astype(np.float32), tws.astype(np.float32))


_C64, _S64, _TWC, _TWS = _dft_consts()


def _body(x_ref, xr0_ref, c_ref, s_ref, twc_ref, tws_ref, out_ref):
    _, C, T = x_ref.shape
    f32 = jnp.float32
    bf16 = jnp.bfloat16
    cm = c_ref[...].astype(bf16)
    sm = s_ref[...].astype(bf16)
    twc = twc_ref[...]
    tws = tws_ref[...]
    k1 = jax.lax.broadcasted_iota(jnp.int32, (_F, _F), 0)
    k2 = jax.lax.broadcasted_iota(jnp.int32, (_F, _F), 1)
    kmask = (k1 + _F * k2 >= 1) & (k1 + _F * k2 <= T // 2)
    ars, ais = [], []
    for i in range(_BB):
        xr = xr0_ref[i].astype(bf16)
        ars.append(jnp.dot(cm, xr, preferred_element_type=f32))
        ais.append(-jnp.dot(sm, xr, preferred_element_type=f32))

    stds = []
    for i in range(_BB):
        xb = x_ref[i]
        s1 = jnp.sum(xb, axis=1, keepdims=True)
        s2 = jnp.sum(xb * xb, axis=1, keepdims=True)
        mean = s1 / T
        var = s2 / T - mean * mean
        std = jnp.sqrt(var)
        stds.append(std)
        a = 1.0 / (std + _EPS)
        bb = -mean * a
        out_ref[i, :C, :] = xb * a + bb

    maskeds = []
    for i in range(_BB):
        br = (ars[i] * twc + ais[i] * tws).astype(bf16)
        bi = (ais[i] * twc - ars[i] * tws).astype(bf16)
        xre = jnp.dot(br, cm, preferred_element_type=f32) + jnp.dot(bi, sm, preferred_element_type=f32)
        xim = jnp.dot(bi, cm, preferred_element_type=f32) - jnp.dot(br, sm, preferred_element_type=f32)
        mag2 = xre * xre + xim * xim
        maskeds.append(jnp.where(kmask, mag2, -1.0))

    vals = [[] for _ in range(_BB)]
    for _ in range(_FFT_TOPK):
        for i in range(_BB):
            m = jnp.max(maskeds[i])
            vals[i].append(m)
            maskeds[i] = jnp.where(maskeds[i] == m, -1.0, maskeds[i])

    for i in range(_BB):
        scale = 1.0 / ((stds[i][0, 0] + _EPS) * (T + 1e-09))
        for j in range(_FFT_TOPK):
            out_ref[i, C + j, :] = jnp.full((T,), jnp.sqrt(vals[i][j]) * scale, f32)


def kernel(x):
    B, C, T = x.shape
    xr0 = x[:, 0, :].reshape(B, _F, _F)
    out = pl.pallas_call(
        _body,
        grid=(B // _BB,),
        in_specs=[
            pl.BlockSpec((_BB, C, T), lambda b: (b, 0, 0)),
            pl.BlockSpec((_BB, _F, _F), lambda b: (b, 0, 0)),
            pl.BlockSpec((_F, _F), lambda b: (0, 0)),
            pl.BlockSpec((_F, _F), lambda b: (0, 0)),
            pl.BlockSpec((_F, _F), lambda b: (0, 0)),
            pl.BlockSpec((_F, _F), lambda b: (0, 0)),
        ],
        out_specs=pl.BlockSpec((_BB, C + _FFT_TOPK, T), lambda b: (b, 0, 0)),
        out_shape=jax.ShapeDtypeStruct((B, C + _FFT_TOPK, T), jnp.float32),
        compiler_params=pltpu.CompilerParams(
            dimension_semantics=("parallel",)),
    )(x, xr0, _C64, _S64, _TWC, _TWS)
    return out


# speedup vs baseline: 1.6970x; 1.0036x over previous
import numpy as np
import jax
import jax.numpy as jnp
from jax.experimental import pallas as pl
from jax.experimental.pallas import tpu as pltpu

_FFT_TOPK = 8
_EPS = 1e-06
_F = 64
_BB = 8


def _dft_consts():
    n = np.arange(_F)
    ang = 2.0 * np.pi * np.outer(n, n) / _F
    c64 = np.cos(ang)
    s64 = np.sin(ang)
    angt = 2.0 * np.pi * np.outer(n, n) / (_F * _F)
    twc = np.cos(angt)
    tws = np.sin(angt)
    return (c64.astype(np.float32), s64.astype(np.float32),
            twc.astype(np.float32), tws.astype(np.float32))


_C64, _S64, _TWC, _TWS = _dft_consts()


def _body(x_ref, xr0_ref, c_ref, s_ref, twc_ref, tws_ref, out_ref):
    _, C, T = x_ref.shape
    f32 = jnp.float32
    bf16 = jnp.bfloat16
    cm = c_ref[...].astype(bf16)
    sm = s_ref[...].astype(bf16)
    twc = twc_ref[...]
    tws = tws_ref[...]
    k1 = jax.lax.broadcasted_iota(jnp.int32, (_F, _F), 0)
    k2 = jax.lax.broadcasted_iota(jnp.int32, (_F, _F), 1)
    kmask = (k1 + _F * k2 >= 1) & (k1 + _F * k2 <= T // 2)
    ars, ais = [], []
    for i in range(_BB):
        xr = xr0_ref[i].astype(bf16)
        ars.append(jnp.dot(cm, xr, preferred_element_type=f32))
        ais.append(-jnp.dot(sm, xr, preferred_element_type=f32))

    stds = []
    for i in range(_BB):
        xb = x_ref[i]
        s1 = jnp.sum(xb, axis=1, keepdims=True)
        s2 = jnp.sum(xb * xb, axis=1, keepdims=True)
        mean = s1 / T
        var = s2 / T - mean * mean
        std = jnp.sqrt(var)
        stds.append(std)
        a = 1.0 / (std + _EPS)
        bb = -mean * a
        out_ref[i, :C, :] = xb * a + bb

    maskeds = []
    for i in range(_BB):
        br = (ars[i] * twc + ais[i] * tws).astype(bf16)
        bi = (ais[i] * twc - ars[i] * tws).astype(bf16)
        xre = jnp.dot(br, cm, preferred_element_type=f32) + jnp.dot(bi, sm, preferred_element_type=f32)
        xim = jnp.dot(bi, cm, preferred_element_type=f32) - jnp.dot(br, sm, preferred_element_type=f32)
        mag2 = xre * xre + xim * xim
        maskeds.append(jnp.where(kmask, mag2, -1.0))

    vals = [[] for _ in range(_BB)]
    for _ in range(_FFT_TOPK):
        for i in range(_BB):
            m = jnp.max(maskeds[i])
            vals[i].append(m)
            maskeds[i] = jnp.where(maskeds[i] == m, -1.0, maskeds[i])

    ri = jax.lax.broadcasted_iota(jnp.int32, (_FFT_TOPK, 1), 0)
    for i in range(_BB):
        scale = 1.0 / ((stds[i][0, 0] + _EPS) * (T + 1e-09))
        v81 = jnp.zeros((_FFT_TOPK, 1), f32)
        for j in range(_FFT_TOPK):
            v81 = jnp.where(ri == j, jnp.sqrt(vals[i][j]) * scale, v81)
        out_ref[i, C:C + _FFT_TOPK, :] = jnp.broadcast_to(v81, (_FFT_TOPK, T))


def kernel(x):
    B, C, T = x.shape
    xr0 = x[:, 0, :].reshape(B, _F, _F)
    out = pl.pallas_call(
        _body,
        grid=(B // _BB,),
        in_specs=[
            pl.BlockSpec((_BB, C, T), lambda b: (b, 0, 0)),
            pl.BlockSpec((_BB, _F, _F), lambda b: (b, 0, 0)),
            pl.BlockSpec((_F, _F), lambda b: (0, 0)),
            pl.BlockSpec((_F, _F), lambda b: (0, 0)),
            pl.BlockSpec((_F, _F), lambda b: (0, 0)),
            pl.BlockSpec((_F, _F), lambda b: (0, 0)),
        ],
        out_specs=pl.BlockSpec((_BB, C + _FFT_TOPK, T), lambda b: (b, 0, 0)),
        out_shape=jax.ShapeDtypeStruct((B, C + _FFT_TOPK, T), jnp.float32),
        compiler_params=pltpu.CompilerParams(
            dimension_semantics=("parallel",)),
    )(x, xr0, _C64, _S64, _TWC, _TWS)
    return out
